# baseline (speedup 1.0000x reference)
import jax
import jax.numpy as jnp
from jax import lax
from jax.experimental import pallas as pl
from jax.experimental.pallas import tpu as pltpu
from jax.experimental.pallas import tpu_sc as plsc

DIM = 64
SCALE = 8.0
LANES = 16

NUM_CORES = 2
NUM_SUBCORES = 16
NUM_WORKERS = NUM_CORES * NUM_SUBCORES

JCHUNK = 2


def _make_body(n_xrows: int, xrow_len: int):
    rows_per_w = n_xrows // NUM_WORKERS
    idx_per_w = rows_per_w * xrow_len
    chunk_idx = JCHUNK * rows_per_w
    n_chunks = xrow_len // JCHUNK
    iblocks = rows_per_w // LANES

    def body(x_hbm, table_hbm, out_hbm, idx_raw, idx_perm,
             gbuf0, gbuf1, obuf0, obuf1, gsem0, gsem1, wsem0, wsem1):
        gbufs = (gbuf0, gbuf1)
        obufs = (obuf0, obuf1)
        gsems = (gsem0, gsem1)
        wsems = (wsem0, wsem1)
        wid = lax.axis_index("s") * NUM_CORES + lax.axis_index("c")
        base = wid * idx_per_w
        col0 = wid * rows_per_w
        pltpu.sync_copy(x_hbm.at[pl.ds(base, idx_per_w)], idx_raw)

        lane = lax.iota(jnp.int32, LANES)

        @pl.loop(0, n_chunks)
        def _perm(c):
            for j in range(JCHUNK):
                for ib in range(iblocks):
                    src = (LANES * ib + lane) * xrow_len + c * JCHUNK + j
                    vals = plsc.load_gather(idx_raw, [src])
                    d0 = c * chunk_idx + j * rows_per_w + LANES * ib
                    idx_perm[pl.ds(d0, LANES)] = vals

        def fire_gather(g, b):
            pltpu.async_copy(
                table_hbm.at[idx_perm.at[pl.ds(g * chunk_idx, chunk_idx)]],
                gbufs[b], gsems[b])

        def wait_gather(b):
            pltpu.make_async_copy(
                table_hbm.at[idx_perm.at[pl.ds(0, chunk_idx)]],
                gbufs[b], gsems[b]).wait()

        def fire_write(g, b):
            pltpu.async_copy(
                obufs[b],
                out_hbm.at[(pl.ds(g * JCHUNK * DIM, JCHUNK * DIM),
                            pl.ds(col0, rows_per_w))],
                wsems[b])

        def wait_write(b):
            pltpu.make_async_copy(
                obufs[b],
                out_hbm.at[(pl.ds(0, JCHUNK * DIM), pl.ds(0, rows_per_w))],
                wsems[b]).wait()

        def transpose_scale(b):
            gbuf = gbufs[b]
            obuf = obufs[b]
            for j in range(JCHUNK):
                @pl.loop(0, DIM, unroll=16)
                def _t(k, gbuf=gbuf, obuf=obuf, j=j):
                    kv = jnp.full((LANES,), k, dtype=jnp.int32)
                    for ib in range(iblocks):
                        rows = j * rows_per_w + LANES * ib + lane
                        vals = plsc.load_gather(gbuf, [rows, kv])
                        obuf[j * DIM + k, pl.ds(LANES * ib, LANES)] = vals * SCALE

        fire_gather(0, 0)
        fire_gather(1, 1)

        @pl.loop(0, n_chunks, step=2)
        def _chunks(gg):
            for b in range(2):
                g = gg + b
                wait_gather(b)

                @pl.when(g >= 2)
                def _():
                    wait_write(b)

                transpose_scale(b)
                fire_write(g, b)

                @pl.when(g + 2 < n_chunks)
                def _():
                    fire_gather(g + 2, b)

        wait_write(0)
        wait_write(1)

    return body


def kernel(x, table):
    n_xrows, xrow_len = x.shape
    xf = x.reshape(-1)
    rows_per_w = n_xrows // NUM_WORKERS
    chunk_idx = JCHUNK * rows_per_w

    mesh = plsc.VectorSubcoreMesh(core_axis_name="c", subcore_axis_name="s")
    out2d = pl.kernel(
        _make_body(n_xrows, xrow_len),
        out_type=jax.ShapeDtypeStruct((xrow_len * DIM, n_xrows), jnp.float32),
        mesh=mesh,
        compiler_params=pltpu.CompilerParams(
            use_tc_tiling_on_sc=False, needs_layout_passes=False,
            disable_bounds_checks=True),
        scratch_types=(
            [pltpu.VMEM((xf.size // NUM_WORKERS,), jnp.int32)] * 2
            + [pltpu.VMEM((chunk_idx, DIM), jnp.float32)] * 2
            + [pltpu.VMEM((JCHUNK * DIM, rows_per_w), jnp.float32)] * 2
            + [pltpu.SemaphoreType.DMA] * 4
        ),
    )(xf, table)
    return out2d.reshape(xrow_len, DIM, n_xrows).transpose(2, 0, 1)


# speedup vs baseline: 1.6211x; 1.6211x over previous
import jax
import jax.numpy as jnp
from jax import lax
from jax.experimental import pallas as pl
from jax.experimental.pallas import tpu as pltpu
from jax.experimental.pallas import tpu_sc as plsc

DIM = 64
SCALE = 8.0
LANES = 16

NUM_CORES = 2
NUM_SUBCORES = 16
NUM_WORKERS = NUM_CORES * NUM_SUBCORES

JCHUNK = 2


def _make_body(n_xrows: int, xrow_len: int):
    rows_per_w = n_xrows // NUM_WORKERS
    idx_per_w = rows_per_w * xrow_len
    chunk_idx = JCHUNK * rows_per_w
    n_chunks = xrow_len // JCHUNK
    iblocks = rows_per_w // LANES

    def body(x_hbm, table_hbm, out_hbm, idx_raw, idx_perm,
             gbuf0, gbuf1, obuf0, obuf1, gsem0, gsem1, wsem0, wsem1):
        gbufs = (gbuf0, gbuf1)
        obufs = (obuf0, obuf1)
        gsems = (gsem0, gsem1)
        wsems = (wsem0, wsem1)
        wid = lax.axis_index("s") * NUM_CORES + lax.axis_index("c")
        base = wid * idx_per_w
        col0 = wid * rows_per_w
        pltpu.sync_copy(x_hbm.at[pl.ds(base, idx_per_w)], idx_raw)

        lane = lax.iota(jnp.int32, LANES)

        @pl.loop(0, n_chunks)
        def _perm(c):
            for j in range(JCHUNK):
                for ib in range(iblocks):
                    src = (LANES * ib + lane) * xrow_len + c * JCHUNK + j
                    vals = plsc.load_gather(idx_raw, [src])
                    d0 = c * chunk_idx + j * rows_per_w + LANES * ib
                    idx_perm[pl.ds(d0, LANES)] = vals

        def fire_gather(g, b):
            pltpu.async_copy(
                table_hbm.at[idx_perm.at[pl.ds(g * chunk_idx, chunk_idx)]],
                gbufs[b], gsems[b])

        def wait_gather(b):
            pltpu.make_async_copy(
                table_hbm.at[idx_perm.at[pl.ds(0, chunk_idx)]],
                gbufs[b], gsems[b]).wait()

        def fire_write(g, b):
            pltpu.async_copy(
                obufs[b].at[(slice(None), pl.ds(0, rows_per_w))],
                out_hbm.at[(pl.ds(g * JCHUNK * DIM, JCHUNK * DIM),
                            pl.ds(col0, rows_per_w))],
                wsems[b])

        def wait_write(b):
            pltpu.make_async_copy(
                obufs[b].at[(slice(None), pl.ds(0, rows_per_w))],
                out_hbm.at[(pl.ds(0, JCHUNK * DIM), pl.ds(0, rows_per_w))],
                wsems[b]).wait()

        def transpose_scale(b):
            gbuf = gbufs[b]
            obuf = obufs[b]

            @pl.loop(0, chunk_idx, unroll=8)
            def _t(r, gbuf=gbuf, obuf=obuf):
                j = lax.shift_right_logical(r, rows_per_w.bit_length() - 1)
                i = lax.bitwise_and(r, rows_per_w - 1)
                iv = jnp.full((LANES,), i, dtype=jnp.int32)
                for kb in range(DIM // LANES):
                    vals = gbuf[r, pl.ds(LANES * kb, LANES)]
                    rowv = j * DIM + LANES * kb + lane
                    plsc.store_scatter(obuf, [rowv, iv], vals * SCALE)

        fire_gather(0, 0)
        fire_gather(1, 1)

        @pl.loop(0, n_chunks, step=2)
        def _chunks(gg):
            for b in range(2):
                g = gg + b
                wait_gather(b)

                @pl.when(g >= 2)
                def _():
                    wait_write(b)

                transpose_scale(b)
                fire_write(g, b)

                @pl.when(g + 2 < n_chunks)
                def _():
                    fire_gather(g + 2, b)

        wait_write(0)
        wait_write(1)

    return body


def kernel(x, table):
    n_xrows, xrow_len = x.shape
    xf = x.reshape(-1)
    rows_per_w = n_xrows // NUM_WORKERS
    chunk_idx = JCHUNK * rows_per_w

    mesh = plsc.VectorSubcoreMesh(core_axis_name="c", subcore_axis_name="s")
    out2d = pl.kernel(
        _make_body(n_xrows, xrow_len),
        out_type=jax.ShapeDtypeStruct((xrow_len * DIM, n_xrows), jnp.float32),
        mesh=mesh,
        compiler_params=pltpu.CompilerParams(
            use_tc_tiling_on_sc=False, needs_layout_passes=False,
            disable_bounds_checks=True),
        scratch_types=(
            [pltpu.VMEM((xf.size // NUM_WORKERS,), jnp.int32)] * 2
            + [pltpu.VMEM((chunk_idx, DIM), jnp.float32)] * 2
            + [pltpu.VMEM((JCHUNK * DIM, rows_per_w + 1), jnp.float32)] * 2
            + [pltpu.SemaphoreType.DMA] * 4
        ),
    )(xf, table)
    return out2d.reshape(xrow_len, DIM, n_xrows).transpose(2, 0, 1)
